# baseline (speedup 1.0000x reference)
import functools

import jax
import jax.numpy as jnp
from jax import lax
from jax.experimental import pallas as pl
from jax.experimental.pallas import tpu as pltpu
from jax.experimental.pallas import tpu_sc as plsc

_N = 10000
_E = 320000
_HID = 128
_NG = 16
_R_MAX = 5.0
_POLY_P = 5.0
_AVG_NEIGH = 32.0

_NC = 2
_NS = 16
_NW = _NC * _NS
_CH = 128
_NCHUNK = 80
_IG = 16
_NGRP = _NCHUNK // _IG
_EPW = _CH * _NCHUNK
_EP = _EPW * _NW
_BE = 2048
_NEB = _EP // _BE
_BN = 1000
_NNB = _N // _BN
_NPT = 640

_mesh = plsc.VectorSubcoreMesh(
    core_axis_name="c", subcore_axis_name="s", num_cores=_NC, num_subcores=_NS)


def _node_slice(sid):
    return sid * _NPT


_SEG = 1024
_NSEG = _EPW // _SEG


def _gather_table_body(table, srcp, dstp, gsrc, gdst,
                       sidx, didx, rows_s, rows_d, gsem):
    wid = lax.axis_index("s") * _NC + lax.axis_index("c")
    base = wid * _EPW

    def seg(sg, carry):
        sb = base + sg * _SEG
        pltpu.sync_copy(srcp.at[pl.ds(sb, _SEG)], sidx)
        pltpu.sync_copy(dstp.at[pl.ds(sb, _SEG)], didx)

        def fire(j, c):
            o = j * _CH
            pltpu.async_copy(table.at[sidx.at[pl.ds(o, _CH)]],
                             rows_s.at[pl.ds(o, _CH)], gsem).wait()
            pltpu.async_copy(table.at[didx.at[pl.ds(o, _CH)]],
                             rows_d.at[pl.ds(o, _CH)], gsem).wait()
            return c

        lax.fori_loop(0, _SEG // _CH, fire, 0)
        pltpu.sync_copy(rows_s, gsrc.at[pl.ds(sb, _SEG)])
        pltpu.sync_copy(rows_d, gdst.at[pl.ds(sb, _SEG)])
        return carry

    lax.fori_loop(0, _NSEG, seg, 0)


_gather_table = functools.partial(
    pl.kernel,
    out_type=(jax.ShapeDtypeStruct((_EP, 4), jnp.float32),
              jax.ShapeDtypeStruct((_EP, 4), jnp.float32)),
    mesh=_mesh,
    compiler_params=pltpu.CompilerParams(use_tc_tiling_on_sc=False),
    scratch_types=(
        pltpu.VMEM((_SEG,), jnp.int32),
        pltpu.VMEM((_SEG,), jnp.int32),
        pltpu.VMEM((_SEG, 4), jnp.float32),
        pltpu.VMEM((_SEG, 4), jnp.float32),
        pltpu.SemaphoreType.DMA,
    ),
)(_gather_table_body)


def _edge_block_math(gs, gd, cs, wr1, wr2, wshl, blk):
    vec = gd[:, :3] - gs[:, :3] + cs
    r2 = jnp.sum(vec * vec, axis=1) + 1e-12
    r = jnp.sqrt(r2)
    inv = 1.0 / r
    x, y, z = vec[:, 0] * inv, vec[:, 1] * inv, vec[:, 2] * inv
    w = [wshl[0, c] for c in range(16)]
    ang = (w[0]
           + w[1] * x + w[2] * y + w[3] * z
           + w[4] * x * y + w[5] * y * z + w[6] * (3.0 * z * z - 1.0)
           + w[7] * x * z + w[8] * (x * x - y * y)
           + w[9] * y * (3.0 * x * x - y * y) + w[10] * x * y * z
           + w[11] * y * (5.0 * z * z - 1.0) + w[12] * z * (5.0 * z * z - 3.0)
           + w[13] * x * (5.0 * z * z - 1.0) + w[14] * z * (x * x - y * y)
           + w[15] * x * (x * x - 3.0 * y * y))
    rr = jnp.maximum(r, 1e-6)
    pref = jnp.sqrt(2.0 / _R_MAX) / rr
    cols = [(pref * jnp.sin(rr * (n * jnp.pi / _R_MAX)))[:, None]
            for n in range(1, 9)]
    rb = jnp.concatenate(cols, axis=1)
    u = jnp.clip(r / _R_MAX, 0.0, 1.0)
    p = _POLY_P
    u2 = u * u
    u4 = u2 * u2
    u5 = u4 * u
    u6 = u5 * u
    u7 = u6 * u
    cut = (1.0 - ((p + 1.0) * (p + 2.0) / 2.0) * u5
           + p * (p + 2.0) * u6 - (p * (p + 1.0) / 2.0) * u7)
    ef = rb * cut[:, None]
    s = jax.nn.silu(jnp.dot(ef, wr1, preferred_element_type=jnp.float32))
    tp = jnp.dot(s, wr2, preferred_element_type=jnp.float32)
    row = blk * _BE + lax.broadcasted_iota(jnp.int32, (_BE,), 0)
    valid = (row < _E).astype(jnp.float32)
    return tp * (ang * valid)[:, None], gs[:, 3]


def _edge_tc0_body(gs_ref, gd_ref, cs_ref, wr1_ref, wr2_ref, wsh_ref,
                   we_ref, msg_ref):
    blk = pl.program_id(0)
    wa, t = _edge_block_math(gs_ref[...], gd_ref[...], cs_ref[...],
                             wr1_ref[...], wr2_ref[...], wsh_ref[...], blk)
    oh = (t[:, None] ==
          lax.broadcasted_iota(jnp.int32, (_BE, 4), 1).astype(jnp.float32)
          ).astype(jnp.float32)
    nf = jnp.dot(oh, we_ref[...], preferred_element_type=jnp.float32)
    msg_ref[...] = nf * wa


def _edge_tc1_body(gs_ref, gd_ref, cs_ref, wr1_ref, wr2_ref, wsh_ref,
                   msg_ref):
    blk = pl.program_id(0)
    wa, _ = _edge_block_math(gs_ref[...], gd_ref[...], cs_ref[...],
                             wr1_ref[...], wr2_ref[...], wsh_ref[...], blk)
    msg_ref[...] = wa


def _edge_specs(with_embed):
    in_specs = [
        pl.BlockSpec((_BE, 4), lambda i: (i, 0)),
        pl.BlockSpec((_BE, 4), lambda i: (i, 0)),
        pl.BlockSpec((_BE, 3), lambda i: (i, 0)),
        pl.BlockSpec((8, 64), lambda i: (0, 0)),
        pl.BlockSpec((64, _HID), lambda i: (0, 0)),
        pl.BlockSpec((1, 16), lambda i: (0, 0)),
    ]
    if with_embed:
        in_specs.append(pl.BlockSpec((4, _HID), lambda i: (0, 0)))
    return in_specs


_edge_tc0 = pl.pallas_call(
    _edge_tc0_body,
    grid=(_NEB,),
    in_specs=_edge_specs(True),
    out_specs=pl.BlockSpec((_BE, _HID), lambda i: (i, 0)),
    out_shape=jax.ShapeDtypeStruct((_EP, _HID), jnp.float32),
)

_edge_tc1 = pl.pallas_call(
    _edge_tc1_body,
    grid=(_NEB,),
    in_specs=_edge_specs(False),
    out_specs=pl.BlockSpec((_BE, _HID), lambda i: (i, 0)),
    out_shape=jax.ShapeDtypeStruct((_EP, _HID), jnp.float32),
)


def _zero_and_dump(sid, cid, zrows, agg_sh, aggp, dump):
    start = _node_slice(sid)

    @pl.when(sid < _NS - 1)
    def _():
        if dump:
            pltpu.sync_copy(agg_sh.at[pl.ds(start, _NPT)],
                            aggp.at[cid, pl.ds(start, _NPT)])
        else:
            pltpu.sync_copy(zrows.at[pl.ds(start, _NPT)],
                            agg_sh.at[pl.ds(start, _NPT)])

    @pl.when(sid == _NS - 1)
    def _():
        if dump:
            pltpu.sync_copy(agg_sh.at[pl.ds(start, _N - 15 * _NPT)],
                            aggp.at[cid, pl.ds(start, _N - 15 * _NPT)])
        else:
            pltpu.sync_copy(zrows.at[pl.ds(start, _N - 15 * _NPT)],
                            agg_sh.at[pl.ds(start, _N - 15 * _NPT)])


def _scatter0_body(msg, dst2d, zrows, aggp,
                   didx2, mrow, agg_sh):
    cid = lax.axis_index("c")
    sid = lax.axis_index("s")
    wid = sid * _NC + cid
    base = wid * _EPW
    _zero_and_dump(sid, cid, zrows, agg_sh, aggp, dump=False)
    plsc.subcore_barrier()

    def group(g, c0):
        pltpu.sync_copy(dst2d.at[pl.ds(wid * _NCHUNK + g * _IG, _IG)], didx2)

        def step(j, c):
            jj = g * _IG + j
            pltpu.sync_copy(msg.at[pl.ds(base + jj * _CH, _CH)], mrow)
            pltpu.sync_copy(mrow, agg_sh.at[didx2.at[j]], add=True)
            return c

        lax.fori_loop(0, _IG, step, 0)
        return c0

    lax.fori_loop(0, _NGRP, group, 0)
    plsc.subcore_barrier()
    _zero_and_dump(sid, cid, zrows, agg_sh, aggp, dump=True)


_scatter0 = functools.partial(
    pl.kernel,
    out_type=jax.ShapeDtypeStruct((_NC, _N, _HID), jnp.float32),
    mesh=_mesh,
    scratch_types=(
        pltpu.VMEM((_IG, _CH), jnp.int32),
        pltpu.VMEM((_CH, _HID), jnp.float32),
        pltpu.VMEM_SHARED((_N, _HID), jnp.float32),
    ),
)(_scatter0_body)


def _gathermul1_body(nf, we, src2d, dst2d, zrows, aggp,
                     sidx2, didx2, grow, wrow, agg_sh, gsem):
    cid = lax.axis_index("c")
    sid = lax.axis_index("s")
    wid = sid * _NC + cid
    base = wid * _EPW
    _zero_and_dump(sid, cid, zrows, agg_sh, aggp, dump=False)
    plsc.subcore_barrier()

    def group(g, c0):
        pltpu.sync_copy(src2d.at[pl.ds(wid * _NCHUNK + g * _IG, _IG)], sidx2)
        pltpu.sync_copy(dst2d.at[pl.ds(wid * _NCHUNK + g * _IG, _IG)], didx2)

        def step(j, c):
            jj = g * _IG + j
            pltpu.async_copy(nf.at[sidx2.at[j]], grow, gsem)
            pltpu.sync_copy(we.at[pl.ds(base + jj * _CH, _CH)], wrow)
            pltpu.make_async_copy(nf.at[pl.ds(0, _CH)], grow, gsem).wait()

            def mul_row(i, c2):
                for k in range(_HID // 16):
                    sl = pl.ds(k * 16, 16)
                    grow[i, sl] = grow[i, sl] * wrow[i, sl]
                return c2

            lax.fori_loop(0, _CH, mul_row, 0)
            pltpu.sync_copy(grow, agg_sh.at[didx2.at[j]], add=True)
            return c

        lax.fori_loop(0, _IG, step, 0)
        return c0

    lax.fori_loop(0, _NGRP, group, 0)
    plsc.subcore_barrier()
    _zero_and_dump(sid, cid, zrows, agg_sh, aggp, dump=True)


_gathermul1 = functools.partial(
    pl.kernel,
    out_type=jax.ShapeDtypeStruct((_NC, _N, _HID), jnp.float32),
    mesh=_mesh,
    scratch_types=(
        pltpu.VMEM((_IG, _CH), jnp.int32),
        pltpu.VMEM((_IG, _CH), jnp.int32),
        pltpu.VMEM((_CH, _HID), jnp.float32),
        pltpu.VMEM((_CH, _HID), jnp.float32),
        pltpu.VMEM_SHARED((_N, _HID), jnp.float32),
        pltpu.SemaphoreType.DMA,
    ),
)(_gathermul1_body)


def _node_common(aggp0, aggp1, nf, oh, wsk, wlin, wp):
    agg = (aggp0 + aggp1) * (1.0 / _AVG_NEIGH)
    h = jnp.dot(agg, wlin, preferred_element_type=jnp.float32)
    sc = jnp.zeros_like(h)
    for t in range(4):
        sc = sc + jnp.dot(nf, wsk[t], preferred_element_type=jnp.float32) \
            * oh[:, t][:, None]
    w1 = jnp.dot(oh, wp[0], preferred_element_type=jnp.float32)
    w2 = jnp.dot(oh, wp[1], preferred_element_type=jnp.float32)
    w3 = jnp.dot(oh, wp[2], preferred_element_type=jnp.float32)
    hp = (w1 + (w2 + w3 * h) * h) * h
    return hp, sc


def _node_tc0_body(aggp_ref, t_ref, b_ref, wemb_ref, wsk_ref, wlin_ref,
                   wp_ref, wprod_ref, wro0_ref, nfo_ref, eseg_ref):
    i = pl.program_id(0)
    t = t_ref[...]
    oh = (t == lax.broadcasted_iota(jnp.int32, (_BN, 4), 1)).astype(jnp.float32)
    nf = jnp.dot(oh, wemb_ref[...], preferred_element_type=jnp.float32)
    hp, sc = _node_common(aggp_ref[0], aggp_ref[1], nf, oh,
                          wsk_ref, wlin_ref[...], wp_ref)
    nfo = jnp.dot(hp, wprod_ref[...], preferred_element_type=jnp.float32) + sc
    nfo_ref[...] = nfo
    e = jnp.dot(nfo, wro0_ref[...], preferred_element_type=jnp.float32)
    ohg = (b_ref[...] == lax.broadcasted_iota(jnp.int32, (_BN, _NG), 1)
           ).astype(jnp.float32)
    eblk = jnp.dot(e.T, ohg, preferred_element_type=jnp.float32)

    @pl.when(i == 0)
    def _():
        eseg_ref[...] = jnp.zeros_like(eseg_ref)

    eseg_ref[...] += eblk


def _node_tc1_body(aggp_ref, t_ref, b_ref, nf_ref, e0_ref, wsk_ref, wlin_ref,
                   wp_ref, wprod_ref, wro1a_ref, wro1b_ref, nfo_ref, eseg_ref):
    i = pl.program_id(0)
    t = t_ref[...]
    oh = (t == lax.broadcasted_iota(jnp.int32, (_BN, 4), 1)).astype(jnp.float32)
    hp, sc = _node_common(aggp_ref[0], aggp_ref[1], nf_ref[...], oh,
                          wsk_ref, wlin_ref[...], wp_ref)
    nfo = jnp.dot(hp, wprod_ref[...], preferred_element_type=jnp.float32) + sc
    nfo_ref[...] = nfo
    ha = jax.nn.silu(jnp.dot(nfo, wro1a_ref[...],
                             preferred_element_type=jnp.float32))
    e = jnp.dot(ha, wro1b_ref[...], preferred_element_type=jnp.float32)
    ohg = (b_ref[...] == lax.broadcasted_iota(jnp.int32, (_BN, _NG), 1)
           ).astype(jnp.float32)
    eblk = jnp.dot(e.T, ohg, preferred_element_type=jnp.float32)

    @pl.when(i == 0)
    def _():
        eseg_ref[...] = e0_ref[...]

    eseg_ref[...] += eblk


_node_tc0 = pl.pallas_call(
    _node_tc0_body,
    grid=(_NNB,),
    in_specs=[
        pl.BlockSpec((_NC, _BN, _HID), lambda i: (0, i, 0)),
        pl.BlockSpec((_BN, 1), lambda i: (i, 0)),
        pl.BlockSpec((_BN, 1), lambda i: (i, 0)),
        pl.BlockSpec((4, _HID), lambda i: (0, 0)),
        pl.BlockSpec((4, _HID, _HID), lambda i: (0, 0, 0)),
        pl.BlockSpec((_HID, _HID), lambda i: (0, 0)),
        pl.BlockSpec((3, 4, _HID), lambda i: (0, 0, 0)),
        pl.BlockSpec((_HID, _HID), lambda i: (0, 0)),
        pl.BlockSpec((_HID, 1), lambda i: (0, 0)),
    ],
    out_specs=[
        pl.BlockSpec((_BN, _HID), lambda i: (i, 0)),
        pl.BlockSpec((1, _NG), lambda i: (0, 0)),
    ],
    out_shape=[
        jax.ShapeDtypeStruct((_N, _HID), jnp.float32),
        jax.ShapeDtypeStruct((1, _NG), jnp.float32),
    ],
)

_node_tc1 = pl.pallas_call(
    _node_tc1_body,
    grid=(_NNB,),
    in_specs=[
        pl.BlockSpec((_NC, _BN, _HID), lambda i: (0, i, 0)),
        pl.BlockSpec((_BN, 1), lambda i: (i, 0)),
        pl.BlockSpec((_BN, 1), lambda i: (i, 0)),
        pl.BlockSpec((_BN, _HID), lambda i: (i, 0)),
        pl.BlockSpec((1, _NG), lambda i: (0, 0)),
        pl.BlockSpec((4, _HID, _HID), lambda i: (0, 0, 0)),
        pl.BlockSpec((_HID, _HID), lambda i: (0, 0)),
        pl.BlockSpec((3, 4, _HID), lambda i: (0, 0, 0)),
        pl.BlockSpec((_HID, _HID), lambda i: (0, 0)),
        pl.BlockSpec((_HID, 16), lambda i: (0, 0)),
        pl.BlockSpec((16, 1), lambda i: (0, 0)),
    ],
    out_specs=[
        pl.BlockSpec((_BN, _HID), lambda i: (i, 0)),
        pl.BlockSpec((1, _NG), lambda i: (0, 0)),
    ],
    out_shape=[
        jax.ShapeDtypeStruct((_N, _HID), jnp.float32),
        jax.ShapeDtypeStruct((1, _NG), jnp.float32),
    ],
)


def kernel(pos, atom_types, edge_index, batch, ptr, cell_shifts, W_embed,
           Wr1, Wr2, wsh, Wsk, Wlin, Wp, Wprod, Wro0, Wro1a, Wro1b):
    src = edge_index[0].astype(jnp.int32)
    dst = edge_index[1].astype(jnp.int32)
    srcp = jnp.zeros((_EP,), jnp.int32).at[:_E].set(src)
    dstp = jnp.zeros((_EP,), jnp.int32).at[:_E].set(dst)
    src2d = srcp.reshape(_EP // _CH, _CH)
    dst2d = dstp.reshape(_EP // _CH, _CH)
    csp = jnp.zeros((_EP, 3), jnp.float32).at[:_E].set(cell_shifts)
    table = jnp.concatenate(
        [pos, atom_types.astype(jnp.float32)[:, None]], axis=1)
    zrows = jnp.zeros((_N, _HID), jnp.float32)
    t2d = atom_types.astype(jnp.int32)[:, None]
    b2d = batch.astype(jnp.int32)[:, None]
    wsh2d = wsh.reshape(2, 1, 16)

    gsrc, gdst = _gather_table(table, srcp, dstp)

    msg0 = _edge_tc0(gsrc, gdst, csp, Wr1[0], Wr2[0], wsh2d[0], W_embed)
    aggp0 = _scatter0(msg0, dst2d, zrows)
    nf1, e0 = _node_tc0(aggp0, t2d, b2d, W_embed, Wsk[0], Wlin[0],
                        Wp[0], Wprod[0], Wro0)

    we1 = _edge_tc1(gsrc, gdst, csp, Wr1[1], Wr2[1], wsh2d[1])
    aggp1 = _gathermul1(nf1, we1, src2d, dst2d, zrows)
    _, e1 = _node_tc1(aggp1, t2d, b2d, nf1, e0, Wsk[1], Wlin[1],
                      Wp[1], Wprod[1], Wro1a, Wro1b)

    return e1.reshape(_NG)
